# baseline (speedup 1.0000x reference)
import functools

import jax
import jax.numpy as jnp
from jax import lax
from jax.experimental import pallas as pl
from jax.experimental.pallas import tpu as pltpu
from jax.experimental.pallas import tpu_sc as plsc
from jax._src.pallas import mpmd
from jax._src.pallas import core as _pc

B = 800
V = 200
D = 98304
NCH = 32
CW = D // NCH
UW = 8
NWIN = V // UW
NBUF = 3
NVEC = B // 16
SPLIT = 560
SROWS = (B - SPLIT) // 2
SBUF = 5

_vmesh = plsc.VectorSubcoreMesh(core_axis_name="c", subcore_axis_name="s")
_smesh = plsc.ScalarSubcoreMesh(axis_name="c")

_scs_smem = _pc.CoreMemorySpace(pltpu.MemorySpace.SMEM, _smesh)
_scs_sems = _pc.CoreMemorySpace(pltpu.MemorySpace.SEMAPHORE, _smesh)
_tec_vmem = _pc.CoreMemorySpace(pltpu.MemorySpace.VMEM, _vmesh)
_tec_smem = _pc.CoreMemorySpace(pltpu.MemorySpace.SMEM, _vmesh)
_tec_sems = _pc.CoreMemorySpace(pltpu.MemorySpace.SEMAPHORE, _vmesh)

_DMA = pltpu.SemaphoreType.DMA.dtype


def _scs_fn(idx_hbm, table_hbm, out_hbm, idx_smem, spbuf, sgsem, swsem,
            idx_raw, uniq, ubuf, cnt, woff, perm, gsem, ssem):
    k = lax.axis_index("c")
    pltpu.sync_copy(idx_hbm, idx_smem)
    base = SPLIT + k * SROWS

    def gather(i, s):
        r = idx_smem[base + i]
        roff = pl.multiple_of(r * NCH, NCH)
        return pltpu.make_async_copy(
            table_hbm.at[pl.ds(roff, NCH)], spbuf.at[s], sgsem.at[s]
        )

    def wout(i, s):
        boff = pl.multiple_of((base + i) * NCH, NCH)
        return pltpu.make_async_copy(
            spbuf.at[s], out_hbm.at[pl.ds(boff, NCH)], swsem.at[s]
        )

    for s in range(SBUF):
        gather(s, s).start()

    def blk_body(blk, carry):
        for j in range(SBUF):
            i = blk * SBUF + j
            gather(i, j).wait()
            wout(i, j).start()
            pj = (j - 1) % SBUF
            ip = i - 1

            @pl.when(ip >= 0)
            def _():
                wout(ip, pj).wait()

                @pl.when(ip + SBUF < SROWS)
                def _():
                    gather(ip + SBUF, pj).start()

        return carry

    lax.fori_loop(0, SROWS // SBUF, blk_body, 0)
    wout(SROWS - 1, (SROWS - 1) % SBUF).wait()


def _tec_fn(idx_hbm, table_hbm, out_hbm, idx_smem, spbuf, sgsem, swsem,
            idx_raw, uniq, ubuf, cnt, woff, perm, gsem, ssem):
    cid = lax.axis_index("c")
    sid = lax.axis_index("s")
    wid = sid * 2 + cid

    pltpu.sync_copy(idx_hbm, idx_raw)

    lanes = lax.broadcasted_iota(jnp.int32, (16,), 0)
    for j in range(13):
        uniq[pl.ds(j * 16, 16)] = (lanes + j * 16) * NCH + wid

    def gather(w, b):
        roff = pl.multiple_of(w * UW, UW)
        return pltpu.make_async_copy(
            table_hbm.at[uniq.at[pl.ds(roff, UW)]], ubuf.at[b], gsem.at[b]
        )

    for b in range(NBUF):
        gather(b, b).start()

    def zero_body(i, carry):
        cnt[i] = 0
        return carry

    lax.fori_loop(0, V, zero_body, 0)

    def count_body(c, carry):
        off = pl.multiple_of(c * 16, 16)
        v = idx_raw[pl.ds(off, 16)]
        for l in range(16):
            t = v[l]
            cnt[t] = cnt[t] + 1
        return carry

    lax.fori_loop(0, NVEC, count_body, 0)

    def scan_body(i, s):
        c = cnt[i]
        cnt[i] = s
        woff[i] = s
        return s + c

    lax.fori_loop(0, V, scan_body, 0)
    cnt[V] = B

    def perm_body(c, carry):
        off = pl.multiple_of(c * 16, 16)
        v = idx_raw[pl.ds(off, 16)]
        base = c * 16
        for l in range(16):
            t = v[l]
            p = woff[t]
            perm[p] = base + l
            woff[t] = p + 1
        return carry

    lax.fori_loop(0, NVEC, perm_body, 0)

    def scatter_window(w, b):
        issued = 0
        for j in range(UW):
            vv = w * UW + j
            src = ubuf.at[b, pl.ds(j, 1), :]

            def sc_body(k, m):
                pos = perm[k]

                @pl.when(pos < SPLIT)
                def _():
                    pltpu.make_async_copy(
                        src,
                        out_hbm.at[pl.ds(pos * NCH + wid, 1)],
                        ssem.at[b],
                    ).start()

                return m + jnp.where(pos < SPLIT, 1, 0)

            issued = lax.fori_loop(cnt[vv], cnt[vv + 1], sc_body, issued)
        return issued

    def drain_window(m, b):
        def drain_body(k, carry):
            pltpu.make_async_copy(
                ubuf.at[b, pl.ds(0, 1), :],
                out_hbm.at[pl.ds(0, 1)],
                ssem.at[b],
            ).wait()
            return carry

        lax.fori_loop(0, m, drain_body, 0)

    prev_m = 0
    for w in range(NWIN):
        b = w % NBUF
        gather(w, b).wait()
        m = scatter_window(w, b)
        if w >= 1:
            p = (w - 1) % NBUF
            drain_window(prev_m, p)
            if w + NBUF - 1 < NWIN:
                gather(w + NBUF - 1, p).start()
        prev_m = m
    drain_window(prev_m, (NWIN - 1) % NBUF)


_call = mpmd.mpmd_map(
    [(_smesh, _scs_fn), (_vmesh, _tec_fn)],
    out_types=[jax.ShapeDtypeStruct((B * NCH, CW), jnp.float32)],
    scratch_types=[
        _scs_smem((B,), jnp.int32),
        pltpu.VMEM_SHARED((SBUF, NCH, CW), jnp.float32),
        _scs_sems((SBUF,), _DMA),
        _scs_sems((SBUF,), _DMA),
        _tec_vmem((B,), jnp.int32),
        _tec_vmem((208,), jnp.int32),
        _tec_vmem((NBUF, UW, CW), jnp.float32),
        _tec_smem((208,), jnp.int32),
        _tec_smem((208,), jnp.int32),
        _tec_smem((B,), jnp.int32),
        _tec_sems((NBUF,), _DMA),
        _tec_sems((NBUF,), _DMA),
    ],
)


def kernel(prefix_tokens, embedding):
    idx = prefix_tokens.reshape(-1).astype(jnp.int32)
    table_r = embedding.reshape(V * NCH, CW)
    (out,) = _call(idx, table_r)
    return out.reshape(prefix_tokens.shape[0], prefix_tokens.shape[1], D)


# speedup vs baseline: 1.0313x; 1.0313x over previous
import functools

import jax
import jax.numpy as jnp
from jax import lax
from jax.experimental import pallas as pl
from jax.experimental.pallas import tpu as pltpu
from jax.experimental.pallas import tpu_sc as plsc
from jax._src.pallas import mpmd
from jax._src.pallas import core as _pc

B = 800
V = 200
D = 98304
NCH = 32
CW = D // NCH
UW = 8
NWIN = V // UW
NBUF = 4
NVEC = B // 16
SPLIT = 728
SROWS = (B - SPLIT) // 2
SBUF = 4

_vmesh = plsc.VectorSubcoreMesh(core_axis_name="c", subcore_axis_name="s")
_smesh = plsc.ScalarSubcoreMesh(axis_name="c")

_scs_smem = _pc.CoreMemorySpace(pltpu.MemorySpace.SMEM, _smesh)
_scs_sems = _pc.CoreMemorySpace(pltpu.MemorySpace.SEMAPHORE, _smesh)
_tec_vmem = _pc.CoreMemorySpace(pltpu.MemorySpace.VMEM, _vmesh)
_tec_smem = _pc.CoreMemorySpace(pltpu.MemorySpace.SMEM, _vmesh)
_tec_sems = _pc.CoreMemorySpace(pltpu.MemorySpace.SEMAPHORE, _vmesh)

_DMA = pltpu.SemaphoreType.DMA.dtype


def _scs_fn(idx_hbm, table_hbm, out_hbm, idx_smem, spbuf, sgsem, swsem,
            idx_raw, uniq, ubuf, cnt, woff, perm, gsem, ssem):
    k = lax.axis_index("c")
    pltpu.sync_copy(idx_hbm, idx_smem)
    base = SPLIT + k * SROWS

    def gather(i, s):
        r = idx_smem[base + i]
        roff = pl.multiple_of(r * NCH, NCH)
        return pltpu.make_async_copy(
            table_hbm.at[pl.ds(roff, NCH)], spbuf.at[s], sgsem.at[s]
        )

    def wout(i, s):
        boff = pl.multiple_of((base + i) * NCH, NCH)
        return pltpu.make_async_copy(
            spbuf.at[s], out_hbm.at[pl.ds(boff, NCH)], swsem.at[s]
        )

    for s in range(SBUF):
        gather(s, s).start()

    def blk_body(blk, carry):
        for j in range(SBUF):
            i = blk * SBUF + j
            gather(i, j).wait()
            wout(i, j).start()
            pj = (j - 1) % SBUF
            ip = i - 1

            @pl.when(ip >= 0)
            def _():
                wout(ip, pj).wait()

                @pl.when(ip + SBUF < SROWS)
                def _():
                    gather(ip + SBUF, pj).start()

        return carry

    lax.fori_loop(0, SROWS // SBUF, blk_body, 0)
    wout(SROWS - 1, (SROWS - 1) % SBUF).wait()


def _tec_fn(idx_hbm, table_hbm, out_hbm, idx_smem, spbuf, sgsem, swsem,
            idx_raw, uniq, ubuf, cnt, woff, perm, gsem, ssem):
    cid = lax.axis_index("c")
    sid = lax.axis_index("s")
    wid = sid * 2 + cid

    pltpu.sync_copy(idx_hbm, idx_raw)

    lanes = lax.broadcasted_iota(jnp.int32, (16,), 0)
    for j in range(13):
        uniq[pl.ds(j * 16, 16)] = (lanes + j * 16) * NCH + wid

    def gather(w, b):
        roff = pl.multiple_of(w * UW, UW)
        return pltpu.make_async_copy(
            table_hbm.at[uniq.at[pl.ds(roff, UW)]], ubuf.at[b], gsem.at[b]
        )

    for b in range(NBUF):
        gather(b, b).start()

    def zero_body(i, carry):
        cnt[i] = 0
        return carry

    lax.fori_loop(0, V, zero_body, 0)

    def count_body(c, carry):
        off = pl.multiple_of(c * 16, 16)
        v = idx_raw[pl.ds(off, 16)]
        for l in range(16):
            t = v[l]
            cnt[t] = cnt[t] + 1
        return carry

    lax.fori_loop(0, NVEC, count_body, 0)

    def scan_body(i, s):
        c = cnt[i]
        cnt[i] = s
        woff[i] = s
        return s + c

    lax.fori_loop(0, V, scan_body, 0)
    cnt[V] = B

    def perm_body(c, carry):
        off = pl.multiple_of(c * 16, 16)
        v = idx_raw[pl.ds(off, 16)]
        base = c * 16
        for l in range(16):
            t = v[l]
            p = woff[t]
            perm[p] = base + l
            woff[t] = p + 1
        return carry

    lax.fori_loop(0, NVEC, perm_body, 0)

    def scatter_window(w, b):
        issued = 0
        for j in range(UW):
            vv = w * UW + j
            src = ubuf.at[b, pl.ds(j, 1), :]

            def sc_body(k, m):
                pos = perm[k]

                @pl.when(pos < SPLIT)
                def _():
                    pltpu.make_async_copy(
                        src,
                        out_hbm.at[pl.ds(pos * NCH + wid, 1)],
                        ssem.at[b],
                    ).start()

                return m + jnp.where(pos < SPLIT, 1, 0)

            issued = lax.fori_loop(cnt[vv], cnt[vv + 1], sc_body, issued)
        return issued

    def drain_window(m, b):
        def drain_body(k, carry):
            pltpu.make_async_copy(
                ubuf.at[b, pl.ds(0, 1), :],
                out_hbm.at[pl.ds(0, 1)],
                ssem.at[b],
            ).wait()
            return carry

        lax.fori_loop(0, m, drain_body, 0)

    prev_m = 0
    for w in range(NWIN):
        b = w % NBUF
        gather(w, b).wait()
        m = scatter_window(w, b)
        if w >= 1:
            p = (w - 1) % NBUF
            drain_window(prev_m, p)
            if w + NBUF - 1 < NWIN:
                gather(w + NBUF - 1, p).start()
        prev_m = m
    drain_window(prev_m, (NWIN - 1) % NBUF)


_call = mpmd.mpmd_map(
    [(_smesh, _scs_fn), (_vmesh, _tec_fn)],
    out_types=[jax.ShapeDtypeStruct((B * NCH, CW), jnp.float32)],
    scratch_types=[
        _scs_smem((B,), jnp.int32),
        pltpu.VMEM_SHARED((SBUF, NCH, CW), jnp.float32),
        _scs_sems((SBUF,), _DMA),
        _scs_sems((SBUF,), _DMA),
        _tec_vmem((B,), jnp.int32),
        _tec_vmem((208,), jnp.int32),
        _tec_vmem((NBUF, UW, CW), jnp.float32),
        _tec_smem((208,), jnp.int32),
        _tec_smem((208,), jnp.int32),
        _tec_smem((B,), jnp.int32),
        _tec_sems((NBUF,), _DMA),
        _tec_sems((NBUF,), _DMA),
    ],
)


def kernel(prefix_tokens, embedding):
    idx = prefix_tokens.reshape(-1).astype(jnp.int32)
    table_r = embedding.reshape(V * NCH, CW)
    (out,) = _call(idx, table_r)
    return out.reshape(prefix_tokens.shape[0], prefix_tokens.shape[1], D)


# speedup vs baseline: 2.1357x; 2.0709x over previous
import functools

import jax
import jax.numpy as jnp
from jax import lax
from jax.experimental import pallas as pl
from jax.experimental.pallas import tpu as pltpu
from jax.experimental.pallas import tpu_sc as plsc

B = 800
V = 200
D = 98304
NCH = 32
CW = D // NCH
UW = 8
NWIN = V // UW
NBUF = 5
NVEC = B // 16

_mesh = plsc.VectorSubcoreMesh(core_axis_name="c", subcore_axis_name="s")


@functools.partial(
    pl.kernel,
    out_type=jax.ShapeDtypeStruct((B, D), jnp.float32),
    mesh=_mesh,
    scratch_types=[
        pltpu.VMEM((B,), jnp.int32),
        pltpu.VMEM((208,), jnp.int32),
        pltpu.VMEM((NBUF, UW, CW), jnp.float32),
        pltpu.SMEM((208,), jnp.int32),
        pltpu.SMEM((208,), jnp.int32),
        pltpu.SMEM((B,), jnp.int32),
        pltpu.SemaphoreType.DMA((NBUF,)),
        pltpu.SemaphoreType.DMA((NBUF,)),
    ],
)
def _sc_dedup(idx_hbm, table_hbm, out_hbm, idx_raw, uniq, ubuf,
              cnt, woff, perm, gsem, ssem):
    cid = lax.axis_index("c")
    sid = lax.axis_index("s")
    wid = sid * 2 + cid
    col = pl.multiple_of(wid * CW, CW)

    pltpu.sync_copy(idx_hbm, idx_raw)

    lanes = lax.broadcasted_iota(jnp.int32, (16,), 0)
    for j in range(13):
        uniq[pl.ds(j * 16, 16)] = (lanes + j * 16) * NCH + wid

    def gather(w, b):
        roff = pl.multiple_of(w * UW, UW)
        return pltpu.make_async_copy(
            table_hbm.at[uniq.at[pl.ds(roff, UW)]], ubuf.at[b], gsem.at[b]
        )

    for b in range(NBUF):
        gather(b, b).start()

    def zero_body(i, carry):
        cnt[i] = 0
        return carry

    lax.fori_loop(0, V, zero_body, 0)

    def count_body(c, carry):
        off = pl.multiple_of(c * 16, 16)
        v = idx_raw[pl.ds(off, 16)]
        for l in range(16):
            t = v[l]
            cnt[t] = cnt[t] + 1
        return carry

    lax.fori_loop(0, NVEC, count_body, 0)

    def scan_body(i, s):
        c = cnt[i]
        cnt[i] = s
        woff[i] = s
        return s + c

    lax.fori_loop(0, V, scan_body, 0)
    cnt[V] = B

    def perm_body(c, carry):
        off = pl.multiple_of(c * 16, 16)
        v = idx_raw[pl.ds(off, 16)]
        base = c * 16
        for l in range(16):
            t = v[l]
            p = woff[t]
            perm[p] = base + l
            woff[t] = p + 1
        return carry

    lax.fori_loop(0, NVEC, perm_body, 0)

    def scatter_window(w, b):
        for j in range(UW):
            vv = w * UW + j
            src = ubuf.at[b, pl.ds(j, 1), :]

            def sc_body(k, carry):
                pos = perm[k]
                pltpu.make_async_copy(
                    src,
                    out_hbm.at[pl.ds(pos, 1), pl.ds(col, CW)],
                    ssem.at[b],
                ).start()
                return carry

            lax.fori_loop(cnt[vv], cnt[vv + 1], sc_body, 0)

        m = cnt[w * UW + UW] - cnt[w * UW]

        def drain_body(k, carry):
            pltpu.make_async_copy(
                ubuf.at[b, pl.ds(0, 1), :],
                out_hbm.at[pl.ds(0, 1), pl.ds(col, CW)],
                ssem.at[b],
            ).wait()
            return carry

        lax.fori_loop(0, m, drain_body, 0)

    for w in range(NWIN):
        b = w % NBUF
        gather(w, b).wait()
        scatter_window(w, b)
        if w + NBUF < NWIN:
            gather(w + NBUF, b).start()


def kernel(prefix_tokens, embedding):
    idx = prefix_tokens.reshape(-1).astype(jnp.int32)
    table_r = embedding.reshape(V * NCH, CW)
    out = _sc_dedup(idx, table_r)
    return out.reshape(prefix_tokens.shape[0], prefix_tokens.shape[1], D)
